# baseline (speedup 1.0000x reference)
import functools

import jax
import jax.numpy as jnp
from jax import lax
from jax.experimental import pallas as pl
from jax.experimental.pallas import tpu as pltpu
from jax.experimental.pallas import tpu_sc as plsc

T = 8192
D = 4096
E = 64
K = 8
LANES = 16

TB = 256


def _matmul_body(x_ref, w_ref, b_ref, o_ref):
    acc = lax.dot_general(
        x_ref[...], w_ref[...],
        dimension_numbers=(((1,), (1,)), ((), ())),
        preferred_element_type=jnp.float32,
    )
    o_ref[...] = acc + b_ref[...]


def _gate_matmul(inp, W, b2d, base, ct):
    nb = base // TB
    return pl.pallas_call(
        _matmul_body,
        grid=(ct // TB,),
        in_specs=[
            pl.BlockSpec((TB, D), lambda i: (i + nb, 0)),
            pl.BlockSpec((E, D), lambda i: (0, 0)),
            pl.BlockSpec((1, E), lambda i: (0, 0)),
        ],
        out_specs=pl.BlockSpec((TB, E), lambda i: (i, 0)),
        out_shape=jax.ShapeDtypeStruct((ct, E), jnp.float32),
    )(inp, W, b2d)


def _merge_top8(ak, av, bk, bv, lane_lt8):
    mk = jnp.where(lane_lt8, ak, lax.rev(bk, (0,)))
    mv = jnp.where(lane_lt8, av, lax.rev(bv, (0,)))
    return plsc.sort_key_val(mk, mv, descending=True)


def _topk_sc(gate, parts=None):
    info = plsc.get_sparse_core_info()
    NC, NS = info.num_cores, info.num_subcores
    NW = NC * NS
    ct = gate.shape[0]
    RPW = ct // NW
    npart = 0 if parts is None else len(parts)
    out_rows = ct if parts is None else (npart + 1) * ct

    mesh = plsc.VectorSubcoreMesh(core_axis_name="c", subcore_axis_name="s")

    @functools.partial(
        pl.kernel,
        out_type=jax.ShapeDtypeStruct((out_rows, E), jnp.float32),
        mesh=mesh,
        scratch_types=[
            pltpu.VMEM((RPW, E), jnp.float32),
            pltpu.VMEM((RPW, E), jnp.float32),
        ]
        + [pltpu.VMEM((RPW, E), jnp.float32) for _ in range(npart)]
        + [pltpu.SemaphoreType.DMA for _ in range(2 * npart)],
        compiler_params=pltpu.CompilerParams(needs_layout_passes=False),
    )
    def k(gate_hbm, *rest):
        part_hbm = rest[:npart]
        out_hbm, g_v, o_v = rest[npart:npart + 3]
        c_v = rest[npart + 3:npart + 3 + npart]
        sems = rest[npart + 3 + npart:]
        wid = lax.axis_index("s") * NC + lax.axis_index("c")
        base = wid * RPW
        ins = [
            pltpu.async_copy(part_hbm[i].at[pl.ds(base, RPW)], c_v[i], sems[i])
            for i in range(npart)
        ]
        pltpu.sync_copy(gate_hbm.at[pl.ds(base, RPW)], g_v)

        lane = lax.iota(jnp.int32, LANES)
        lane_lt8 = lane < K
        zeros16 = jnp.zeros((LANES,), jnp.float32)

        @plsc.parallel_loop(0, RPW, unroll=2)
        def row_body(r):
            sk = []
            sv = []
            for c in range(E // LANES):
                g = g_v[r, pl.ds(c * LANES, LANES)]
                k_, v_ = plsc.sort_key_val(g, lane + c * LANES, descending=True)
                sk.append(k_)
                sv.append(v_)
            k01, v01 = _merge_top8(sk[0], sv[0], sk[1], sv[1], lane_lt8)
            k23, v23 = _merge_top8(sk[2], sv[2], sk[3], sv[3], lane_lt8)
            fk, fv = _merge_top8(k01, v01, k23, v23, lane_lt8)

            m = jnp.max(fk)
            e = jnp.where(lane_lt8, jnp.exp(fk - m), 0.0)
            s = jnp.broadcast_to(jnp.sum(e), (LANES,))
            probs = e / s

            for c in range(E // LANES):
                o_v[r, pl.ds(c * LANES, LANES)] = zeros16
            rows = jnp.full((LANES,), r, jnp.int32)
            plsc.store_scatter(o_v, [rows, fv], probs, mask=lane_lt8)

        outs = []
        for i in range(npart):
            ins[i].wait()
            outs.append(
                pltpu.async_copy(
                    c_v[i], out_hbm.at[pl.ds(i * ct + base, RPW)],
                    sems[npart + i],
                )
            )
        pltpu.sync_copy(o_v, out_hbm.at[pl.ds(npart * ct + base, RPW)])
        for o in outs:
            o.wait()

    if parts is None:
        return k(gate)
    return k(gate, *parts)


NCHUNK = 2


@jax.jit
def kernel(inp, W, b):
    b2d = b.reshape(1, E)
    ct = T // NCHUNK
    parts = []
    for i in range(NCHUNK - 1):
        gate = _gate_matmul(inp, W, b2d, i * ct, ct)
        parts.append(_topk_sc(gate))
    gate = _gate_matmul(inp, W, b2d, (NCHUNK - 1) * ct, ct)
    return _topk_sc(gate, parts=tuple(parts))


# speedup vs baseline: 1.0931x; 1.0931x over previous
import functools

import jax
import jax.numpy as jnp
from jax import lax
from jax.experimental import pallas as pl
from jax.experimental.pallas import tpu as pltpu
from jax.experimental.pallas import tpu_sc as plsc

T = 8192
D = 4096
E = 64
K = 8
LANES = 16

TB = 512


def _matmul_body(x_ref, w_ref, b_ref, o_ref):
    acc = lax.dot_general(
        x_ref[...], w_ref[...],
        dimension_numbers=(((1,), (1,)), ((), ())),
        preferred_element_type=jnp.float32,
    )
    o_ref[...] = acc + b_ref[...]


def _gate_matmul(inp, W, b2d, base, ct):
    nb = base // TB
    return pl.pallas_call(
        _matmul_body,
        grid=(ct // TB,),
        in_specs=[
            pl.BlockSpec((TB, D), lambda i: (i + nb, 0)),
            pl.BlockSpec((E, D), lambda i: (0, 0)),
            pl.BlockSpec((1, E), lambda i: (0, 0)),
        ],
        out_specs=pl.BlockSpec((TB, E), lambda i: (i, 0)),
        out_shape=jax.ShapeDtypeStruct((ct, E), jnp.float32),
    )(inp, W, b2d)


def _merge_top8(ak, av, bk, bv, lane_lt8):
    mk = jnp.where(lane_lt8, ak, lax.rev(bk, (0,)))
    mv = jnp.where(lane_lt8, av, lax.rev(bv, (0,)))
    return plsc.sort_key_val(mk, mv, descending=True)


def _topk_sc(gate, parts=None):
    info = plsc.get_sparse_core_info()
    NC, NS = info.num_cores, info.num_subcores
    NW = NC * NS
    ct = gate.shape[0]
    RPW = ct // NW
    npart = 0 if parts is None else len(parts)
    out_rows = ct if parts is None else (npart + 1) * ct

    mesh = plsc.VectorSubcoreMesh(core_axis_name="c", subcore_axis_name="s")

    @functools.partial(
        pl.kernel,
        out_type=jax.ShapeDtypeStruct((out_rows, E), jnp.float32),
        mesh=mesh,
        scratch_types=[
            pltpu.VMEM((RPW, E), jnp.float32),
            pltpu.VMEM((RPW, E), jnp.float32),
        ]
        + [pltpu.VMEM((RPW, E), jnp.float32) for _ in range(npart)]
        + [pltpu.SemaphoreType.DMA for _ in range(2 * npart)],
        compiler_params=pltpu.CompilerParams(needs_layout_passes=False),
    )
    def k(gate_hbm, *rest):
        part_hbm = rest[:npart]
        out_hbm, g_v, o_v = rest[npart:npart + 3]
        c_v = rest[npart + 3:npart + 3 + npart]
        sems = rest[npart + 3 + npart:]
        wid = lax.axis_index("s") * NC + lax.axis_index("c")
        base = wid * RPW
        ins = [
            pltpu.async_copy(part_hbm[i].at[pl.ds(base, RPW)], c_v[i], sems[i])
            for i in range(npart)
        ]
        pltpu.sync_copy(gate_hbm.at[pl.ds(base, RPW)], g_v)

        lane = lax.iota(jnp.int32, LANES)
        lane_lt8 = lane < K
        zeros16 = jnp.zeros((LANES,), jnp.float32)

        @plsc.parallel_loop(0, RPW, unroll=2)
        def row_body(r):
            sk = []
            sv = []
            for c in range(E // LANES):
                g = g_v[r, pl.ds(c * LANES, LANES)]
                k_, v_ = plsc.sort_key_val(g, lane + c * LANES, descending=True)
                sk.append(k_)
                sv.append(v_)
            k01, v01 = _merge_top8(sk[0], sv[0], sk[1], sv[1], lane_lt8)
            k23, v23 = _merge_top8(sk[2], sv[2], sk[3], sv[3], lane_lt8)
            fk, fv = _merge_top8(k01, v01, k23, v23, lane_lt8)

            m = jnp.max(fk)
            e = jnp.where(lane_lt8, jnp.exp(fk - m), 0.0)
            s = jnp.broadcast_to(jnp.sum(e), (LANES,))
            probs = e / s

            for c in range(E // LANES):
                o_v[r, pl.ds(c * LANES, LANES)] = zeros16
            rows = jnp.full((LANES,), r, jnp.int32)
            plsc.store_scatter(o_v, [rows, fv], probs, mask=lane_lt8)

        outs = []
        for i in range(npart):
            ins[i].wait()
            outs.append(
                pltpu.async_copy(
                    c_v[i], out_hbm.at[pl.ds(i * ct + base, RPW)],
                    sems[npart + i],
                )
            )
        pltpu.sync_copy(o_v, out_hbm.at[pl.ds(npart * ct + base, RPW)])
        for o in outs:
            o.wait()

    if parts is None:
        return k(gate)
    return k(gate, *parts)


NCHUNK = 2


@jax.jit
def kernel(inp, W, b):
    b2d = b.reshape(1, E)
    ct = T // NCHUNK
    parts = []
    for i in range(NCHUNK - 1):
        gate = _gate_matmul(inp, W, b2d, i * ct, ct)
        parts.append(_topk_sc(gate))
    gate = _gate_matmul(inp, W, b2d, (NCHUNK - 1) * ct, ct)
    return _topk_sc(gate, parts=tuple(parts))
